# Initial kernel scaffold; baseline (speedup 1.0000x reference)
import jax
import jax.numpy as jnp
from jax.experimental import pallas as pl


def kernel(x, edge_index, batch, W1_init, W1_root, b1, W2_init, W2_root, b2, W3_init, W3_root, b3):
    raise NotImplementedError("write your pallas kernel here")


# speedup vs baseline: 8.5572x; 8.5572x over previous
import functools

import jax
import jax.numpy as jnp
from jax import lax
from jax.experimental import pallas as pl
from jax.experimental.pallas import tpu as pltpu
from jax.experimental.pallas import tpu_sc as plsc

N_NODES = 10000
N_PAD = 10240
E = 320000
D_IN = 128
D_HID = 128
D_OUT = 64

NC, NS = 2, 16
NW = NC * NS
E_PER_W = E // NW
CHUNK = 80
N_CHUNKS = E_PER_W // CHUNK
ROWS_PER_TILE = N_PAD // NS

_MESH = plsc.VectorSubcoreMesh(core_axis_name="c", subcore_axis_name="s")
_SC_PARAMS = pltpu.CompilerParams(use_tc_tiling_on_sc=False)


@functools.partial(
    pl.kernel,
    out_type=jax.ShapeDtypeStruct((NC, N_PAD, 16), jnp.float32),
    mesh=_MESH,
    scratch_types=[
        pltpu.VMEM((CHUNK,), jnp.int32),
        pltpu.VMEM((CHUNK, 16), jnp.float32),
        pltpu.VMEM_SHARED((N_PAD, 16), jnp.float32),
    ],
    compiler_params=_SC_PARAMS,
)
def _deg_kernel(col_hbm, ones_hbm, zeros_hbm, out_hbm, cidx, ones_v, acc):
    c = lax.axis_index("c")
    s = lax.axis_index("s")
    wid = c * NS + s
    e0 = wid * E_PER_W
    pltpu.sync_copy(zeros_hbm, acc.at[pl.ds(s * ROWS_PER_TILE, ROWS_PER_TILE)])
    pltpu.sync_copy(ones_hbm, ones_v)
    plsc.subcore_barrier()

    def body(i, carry):
        base = e0 + i * CHUNK
        pltpu.sync_copy(col_hbm.at[pl.ds(base, CHUNK)], cidx)
        pltpu.sync_copy(ones_v, acc.at[cidx], add=True)
        return carry

    lax.fori_loop(0, N_CHUNKS, body, 0)
    plsc.subcore_barrier()
    pltpu.sync_copy(
        acc.at[pl.ds(s * ROWS_PER_TILE, ROWS_PER_TILE)],
        out_hbm.at[c, pl.ds(s * ROWS_PER_TILE, ROWS_PER_TILE)],
    )


def _make_edge_scatter(d):
    @functools.partial(
        pl.kernel,
        out_type=jax.ShapeDtypeStruct((NC, N_PAD, d), jnp.float32),
        mesh=_MESH,
        scratch_types=[
            pltpu.VMEM((CHUNK,), jnp.int32),
            pltpu.VMEM((CHUNK,), jnp.int32),
            pltpu.VMEM((CHUNK, d), jnp.float32),
            pltpu.VMEM_SHARED((N_PAD, d), jnp.float32),
            pltpu.SemaphoreType.DMA,
        ],
        compiler_params=_SC_PARAMS,
    )
    def edge_kernel(g_hbm, row_hbm, col_hbm, zeros_hbm, out_hbm,
                    ridx, cidx, rows, acc, gsem):
        c = lax.axis_index("c")
        s = lax.axis_index("s")
        wid = c * NS + s
        e0 = wid * E_PER_W
        pltpu.sync_copy(zeros_hbm,
                        acc.at[pl.ds(s * ROWS_PER_TILE, ROWS_PER_TILE)])
        plsc.subcore_barrier()

        def body(i, carry):
            base = e0 + i * CHUNK
            pltpu.sync_copy(row_hbm.at[pl.ds(base, CHUNK)], ridx)
            pltpu.sync_copy(col_hbm.at[pl.ds(base, CHUNK)], cidx)
            pltpu.async_copy(g_hbm.at[ridx], rows, gsem).wait()
            pltpu.sync_copy(rows, acc.at[cidx], add=True)
            return carry

        lax.fori_loop(0, N_CHUNKS, body, 0)
        plsc.subcore_barrier()
        pltpu.sync_copy(
            acc.at[pl.ds(s * ROWS_PER_TILE, ROWS_PER_TILE)],
            out_hbm.at[c, pl.ds(s * ROWS_PER_TILE, ROWS_PER_TILE)],
        )

    return edge_kernel


_edge_scatter_128 = _make_edge_scatter(D_HID)
_edge_scatter_64 = _make_edge_scatter(D_OUT)


_BLK = 400


def _dis_body(degp_ref, dis_ref):
    deg = degp_ref[0, :, 0:1] + degp_ref[1, :, 0:1]
    dis_ref[...] = jnp.where(deg > 0, lax.rsqrt(deg), 0.0)


def _dis_kernel(deg_parts):
    return pl.pallas_call(
        _dis_body,
        grid=(8,),
        in_specs=[pl.BlockSpec((NC, N_PAD // 8, 16), lambda i: (0, i, 0))],
        out_specs=pl.BlockSpec((N_PAD // 8, 1), lambda i: (i, 0)),
        out_shape=jax.ShapeDtypeStruct((N_PAD, 1), jnp.float32),
    )(deg_parts)


def _pre_body(x_ref, wi_ref, wr_ref, b_ref, dis_ref, g_ref, r_ref):
    x = x_ref[...]
    h = jnp.dot(x, wi_ref[...], preferred_element_type=jnp.float32)
    g_ref[...] = h * dis_ref[...]
    r_ref[...] = (
        jnp.dot(x, wr_ref[...], preferred_element_type=jnp.float32)
        + b_ref[...]
    )


def _pre_kernel(x, wi, wr, b, dis):
    d_in, d = wi.shape
    return pl.pallas_call(
        _pre_body,
        grid=(N_NODES // _BLK,),
        in_specs=[
            pl.BlockSpec((_BLK, d_in), lambda i: (i, 0)),
            pl.BlockSpec((d_in, d), lambda i: (0, 0)),
            pl.BlockSpec((d_in, d), lambda i: (0, 0)),
            pl.BlockSpec((1, d), lambda i: (0, 0)),
            pl.BlockSpec((_BLK, 1), lambda i: (i, 0)),
        ],
        out_specs=[
            pl.BlockSpec((_BLK, d), lambda i: (i, 0)),
            pl.BlockSpec((_BLK, d), lambda i: (i, 0)),
        ],
        out_shape=[
            jax.ShapeDtypeStruct((N_NODES, d), jnp.float32),
            jax.ShapeDtypeStruct((N_NODES, d), jnp.float32),
        ],
    )(x, wi, wr, b, dis)


def _make_post_body(final):
    def body(parts_ref, r_ref, dis_ref, out_ref):
        z = (parts_ref[0] + parts_ref[1]) * dis_ref[...] + r_ref[...]
        z = jnp.maximum(z, 0.0)
        if final:
            z = jax.nn.sigmoid(z)
        out_ref[...] = z
    return body


def _post_kernel(parts, r, dis, final):
    d = r.shape[1]
    return pl.pallas_call(
        _make_post_body(final),
        grid=(N_NODES // _BLK,),
        in_specs=[
            pl.BlockSpec((NC, _BLK, d), lambda i: (0, i, 0)),
            pl.BlockSpec((_BLK, d), lambda i: (i, 0)),
            pl.BlockSpec((_BLK, 1), lambda i: (i, 0)),
        ],
        out_specs=pl.BlockSpec((_BLK, d), lambda i: (i, 0)),
        out_shape=jax.ShapeDtypeStruct((N_NODES, d), jnp.float32),
    )(parts, r, dis)


def kernel(x, edge_index, batch, W1_init, W1_root, b1,
           W2_init, W2_root, b2, W3_init, W3_root, b3):
    row = edge_index[0]
    col = edge_index[1]
    zeros128 = jnp.zeros((ROWS_PER_TILE, D_HID), jnp.float32)
    zeros64 = jnp.zeros((ROWS_PER_TILE, D_OUT), jnp.float32)
    zeros16 = jnp.zeros((ROWS_PER_TILE, 16), jnp.float32)
    ones16 = jnp.ones((CHUNK, 16), jnp.float32)

    deg_parts = _deg_kernel(col, ones16, zeros16)
    dis = _dis_kernel(deg_parts)

    h = x
    for (wi, wr, b, scat, zeros, final) in (
        (W1_init, W1_root, b1, _edge_scatter_128, zeros128, False),
        (W2_init, W2_root, b2, _edge_scatter_128, zeros128, False),
        (W3_init, W3_root, b3, _edge_scatter_64, zeros64, True),
    ):
        g, r = _pre_kernel(h, wi, wr, jnp.reshape(b, (1, -1)), dis)
        parts = scat(g, row, col, zeros)
        h = _post_kernel(parts, r, dis, final)
    return h


# speedup vs baseline: 18.6458x; 2.1789x over previous
import functools

import jax
import jax.numpy as jnp
from jax import lax
from jax.experimental import pallas as pl
from jax.experimental.pallas import tpu as pltpu
from jax.experimental.pallas import tpu_sc as plsc

N_NODES = 10000
N_PAD = 10240
E = 320000
D_IN = 128
D_HID = 128
D_OUT = 64

NC, NS = 2, 16
NW = NC * NS
E_PER_W = E // NW
CHUNK = 80
N_CHUNKS = E_PER_W // CHUNK
ROWS_PER_TILE = N_PAD // NS

_MESH = plsc.VectorSubcoreMesh(core_axis_name="c", subcore_axis_name="s")
_SC_PARAMS = pltpu.CompilerParams(use_tc_tiling_on_sc=False)


NB = 4
GA = 2
NFULL = (N_CHUNKS // NB) * NB


@functools.partial(
    pl.kernel,
    out_type=jax.ShapeDtypeStruct((NC, N_PAD, 16), jnp.float32),
    mesh=_MESH,
    scratch_types=(
        [pltpu.VMEM((CHUNK,), jnp.int32) for _ in range(NB)]
        + [pltpu.VMEM((CHUNK, 16), jnp.float32),
           pltpu.VMEM_SHARED((N_PAD, 16), jnp.float32)]
        + [pltpu.SemaphoreType.DMA for _ in range(NB)]
    ),
    compiler_params=_SC_PARAMS,
)
def _deg_kernel(edge_hbm, ones_hbm, zeros_hbm, out_hbm, *scr):
    cidx = scr[0:NB]
    ones_v, acc = scr[NB], scr[NB + 1]
    isem = scr[NB + 2:NB + 2 + NB]
    c = lax.axis_index("c")
    s = lax.axis_index("s")
    wid = c * NS + s
    e0 = wid * E_PER_W
    pltpu.sync_copy(zeros_hbm, acc.at[pl.ds(s * ROWS_PER_TILE, ROWS_PER_TILE)])
    pltpu.sync_copy(ones_hbm, ones_v)
    plsc.subcore_barrier()

    def cp_idx(k, q):
        return pltpu.async_copy(
            edge_hbm.at[1, pl.ds(e0 + k * CHUNK, CHUNK)], cidx[q], isem[q])

    def wait_idx(k, q):
        pltpu.make_async_copy(
            edge_hbm.at[1, pl.ds(e0 + k * CHUNK, CHUNK)],
            cidx[q], isem[q]).wait()

    for q in range(NB):
        cp_idx(q, q)

    def body(i0, carry):
        for q in range(NB):
            i = i0 * NB + q
            wait_idx(i, q)
            pltpu.sync_copy(ones_v, acc.at[cidx[q]], add=True)

            @pl.when(i + NB < N_CHUNKS)
            def _():
                cp_idx(i + NB, q)
        return carry

    lax.fori_loop(0, NFULL // NB, body, 0)
    for i in range(NFULL, N_CHUNKS):
        q = i % NB
        wait_idx(i, q)
        pltpu.sync_copy(ones_v, acc.at[cidx[q]], add=True)

    plsc.subcore_barrier()
    pltpu.sync_copy(
        acc.at[pl.ds(s * ROWS_PER_TILE, ROWS_PER_TILE)],
        out_hbm.at[c, pl.ds(s * ROWS_PER_TILE, ROWS_PER_TILE)],
    )


def _make_edge_scatter(d):
    @functools.partial(
        pl.kernel,
        out_type=jax.ShapeDtypeStruct((NC, N_PAD, d), jnp.float32),
        mesh=_MESH,
        scratch_types=(
            [pltpu.VMEM((2, CHUNK), jnp.int32) for _ in range(NB)]
            + [pltpu.VMEM((CHUNK, d), jnp.float32) for _ in range(NB)]
            + [pltpu.VMEM_SHARED((N_PAD, d), jnp.float32)]
            + [pltpu.SemaphoreType.DMA for _ in range(2 * NB)]
        ),
        compiler_params=_SC_PARAMS,
    )
    def edge_kernel(g_hbm, edge_hbm, zeros_hbm, out_hbm, *scr):
        idxb = scr[0:NB]
        rows = scr[NB:2 * NB]
        acc = scr[2 * NB]
        isem = scr[2 * NB + 1:3 * NB + 1]
        gsem = scr[3 * NB + 1:4 * NB + 1]
        c = lax.axis_index("c")
        s = lax.axis_index("s")
        wid = c * NS + s
        e0 = wid * E_PER_W
        pltpu.sync_copy(zeros_hbm,
                        acc.at[pl.ds(s * ROWS_PER_TILE, ROWS_PER_TILE)])
        plsc.subcore_barrier()

        def cp_idx(k, q):
            return pltpu.async_copy(
                edge_hbm.at[:, pl.ds(e0 + k * CHUNK, CHUNK)], idxb[q], isem[q])

        def wait_idx(k, q):
            pltpu.make_async_copy(
                edge_hbm.at[:, pl.ds(e0 + k * CHUNK, CHUNK)],
                idxb[q], isem[q]).wait()

        def start_gather(q):
            return pltpu.async_copy(g_hbm.at[idxb[q].at[0]], rows[q], gsem[q])

        def wait_gather(q):
            pltpu.make_async_copy(
                g_hbm.at[idxb[q].at[0]], rows[q], gsem[q]).wait()

        for q in range(NB):
            cp_idx(q, q)
        for q in range(GA):
            wait_idx(q, q)
            start_gather(q)

        def body(i0, carry):
            for q in range(NB):
                i = i0 * NB + q
                wait_gather(q)
                pltpu.sync_copy(rows[q], acc.at[idxb[q].at[1]], add=True)

                @pl.when(i + NB < N_CHUNKS)
                def _():
                    cp_idx(i + NB, q)

                qg = (q + GA) % NB

                @pl.when(i + GA < N_CHUNKS)
                def _():
                    wait_idx(i + GA, qg)
                    start_gather(qg)
            return carry

        lax.fori_loop(0, NFULL // NB, body, 0)
        for i in range(NFULL, N_CHUNKS):
            q = i % NB
            wait_gather(q)
            pltpu.sync_copy(rows[q], acc.at[idxb[q].at[1]], add=True)

        plsc.subcore_barrier()
        pltpu.sync_copy(
            acc.at[pl.ds(s * ROWS_PER_TILE, ROWS_PER_TILE)],
            out_hbm.at[c, pl.ds(s * ROWS_PER_TILE, ROWS_PER_TILE)],
        )

    return edge_kernel


_edge_scatter_128 = _make_edge_scatter(D_HID)
_edge_scatter_64 = _make_edge_scatter(D_OUT)


_BLK = 400


def _dis_body(degp_ref, dis_ref):
    deg = degp_ref[0, :, 0:1] + degp_ref[1, :, 0:1]
    dis_ref[...] = jnp.where(deg > 0, lax.rsqrt(deg), 0.0)


def _dis_kernel(deg_parts):
    return pl.pallas_call(
        _dis_body,
        grid=(8,),
        in_specs=[pl.BlockSpec((NC, N_PAD // 8, 16), lambda i: (0, i, 0))],
        out_specs=pl.BlockSpec((N_PAD // 8, 1), lambda i: (i, 0)),
        out_shape=jax.ShapeDtypeStruct((N_PAD, 1), jnp.float32),
    )(deg_parts)


def _pre_body(x_ref, wi_ref, wr_ref, b_ref, dis_ref, g_ref, r_ref):
    x = x_ref[...]
    h = jnp.dot(x, wi_ref[...], preferred_element_type=jnp.float32)
    g_ref[...] = h * dis_ref[...]
    r_ref[...] = (
        jnp.dot(x, wr_ref[...], preferred_element_type=jnp.float32)
        + b_ref[...]
    )


def _pre_kernel(x, wi, wr, b, dis):
    d_in, d = wi.shape
    return pl.pallas_call(
        _pre_body,
        grid=(N_NODES // _BLK,),
        in_specs=[
            pl.BlockSpec((_BLK, d_in), lambda i: (i, 0)),
            pl.BlockSpec((d_in, d), lambda i: (0, 0)),
            pl.BlockSpec((d_in, d), lambda i: (0, 0)),
            pl.BlockSpec((1, d), lambda i: (0, 0)),
            pl.BlockSpec((_BLK, 1), lambda i: (i, 0)),
        ],
        out_specs=[
            pl.BlockSpec((_BLK, d), lambda i: (i, 0)),
            pl.BlockSpec((_BLK, d), lambda i: (i, 0)),
        ],
        out_shape=[
            jax.ShapeDtypeStruct((N_NODES, d), jnp.float32),
            jax.ShapeDtypeStruct((N_NODES, d), jnp.float32),
        ],
    )(x, wi, wr, b, dis)


def _make_post_body(final):
    def body(parts_ref, r_ref, dis_ref, out_ref):
        z = (parts_ref[0] + parts_ref[1]) * dis_ref[...] + r_ref[...]
        z = jnp.maximum(z, 0.0)
        if final:
            z = jax.nn.sigmoid(z)
        out_ref[...] = z
    return body


def _post_kernel(parts, r, dis, final):
    d = r.shape[1]
    return pl.pallas_call(
        _make_post_body(final),
        grid=(N_NODES // _BLK,),
        in_specs=[
            pl.BlockSpec((NC, _BLK, d), lambda i: (0, i, 0)),
            pl.BlockSpec((_BLK, d), lambda i: (i, 0)),
            pl.BlockSpec((_BLK, 1), lambda i: (i, 0)),
        ],
        out_specs=pl.BlockSpec((_BLK, d), lambda i: (i, 0)),
        out_shape=jax.ShapeDtypeStruct((N_NODES, d), jnp.float32),
    )(parts, r, dis)


def kernel(x, edge_index, batch, W1_init, W1_root, b1,
           W2_init, W2_root, b2, W3_init, W3_root, b3):
    zeros128 = jnp.zeros((ROWS_PER_TILE, D_HID), jnp.float32)
    zeros64 = jnp.zeros((ROWS_PER_TILE, D_OUT), jnp.float32)
    zeros16 = jnp.zeros((ROWS_PER_TILE, 16), jnp.float32)
    ones16 = jnp.ones((CHUNK, 16), jnp.float32)

    deg_parts = _deg_kernel(edge_index, ones16, zeros16)
    dis = _dis_kernel(deg_parts)

    h = x
    for (wi, wr, b, scat, zeros, final) in (
        (W1_init, W1_root, b1, _edge_scatter_128, zeros128, False),
        (W2_init, W2_root, b2, _edge_scatter_128, zeros128, False),
        (W3_init, W3_root, b3, _edge_scatter_64, zeros64, True),
    ):
        g, r = _pre_kernel(h, wi, wr, jnp.reshape(b, (1, -1)), dis)
        parts = scat(g, edge_index, zeros)
        h = _post_kernel(parts, r, dis, final)
    return h


# speedup vs baseline: 19.2569x; 1.0328x over previous
import functools

import jax
import jax.numpy as jnp
from jax import lax
from jax.experimental import pallas as pl
from jax.experimental.pallas import tpu as pltpu
from jax.experimental.pallas import tpu_sc as plsc

N_NODES = 10000
N_PAD = 10240
E = 320000
D_IN = 128
D_HID = 128
D_OUT = 64

NC, NS = 2, 16
NW = NC * NS
E_PER_W = E // NW
CHUNK = 80
N_CHUNKS = E_PER_W // CHUNK
ROWS_PER_TILE = N_PAD // NS

_MESH = plsc.VectorSubcoreMesh(core_axis_name="c", subcore_axis_name="s")
_SC_PARAMS = pltpu.CompilerParams(use_tc_tiling_on_sc=False)


NB = 4
GA = 2
NFULL = (N_CHUNKS // NB) * NB


@functools.partial(
    pl.kernel,
    out_type=jax.ShapeDtypeStruct((NC, N_PAD, 16), jnp.float32),
    mesh=_MESH,
    scratch_types=(
        [pltpu.VMEM((CHUNK,), jnp.int32) for _ in range(NB)]
        + [pltpu.VMEM((CHUNK, 16), jnp.float32),
           pltpu.VMEM_SHARED((N_PAD, 16), jnp.float32)]
        + [pltpu.SemaphoreType.DMA for _ in range(NB)]
    ),
    compiler_params=_SC_PARAMS,
)
def _deg_kernel(edge_hbm, ones_hbm, zeros_hbm, out_hbm, *scr):
    cidx = scr[0:NB]
    ones_v, acc = scr[NB], scr[NB + 1]
    isem = scr[NB + 2:NB + 2 + NB]
    c = lax.axis_index("c")
    s = lax.axis_index("s")
    wid = c * NS + s
    e0 = wid * E_PER_W
    pltpu.sync_copy(zeros_hbm, acc.at[pl.ds(s * ROWS_PER_TILE, ROWS_PER_TILE)])
    pltpu.sync_copy(ones_hbm, ones_v)
    plsc.subcore_barrier()

    def cp_idx(k, q):
        return pltpu.async_copy(
            edge_hbm.at[1, pl.ds(e0 + k * CHUNK, CHUNK)], cidx[q], isem[q])

    def wait_idx(k, q):
        pltpu.make_async_copy(
            edge_hbm.at[1, pl.ds(e0 + k * CHUNK, CHUNK)],
            cidx[q], isem[q]).wait()

    for q in range(NB):
        cp_idx(q, q)

    def body(i0, carry):
        for q in range(NB):
            i = i0 * NB + q
            wait_idx(i, q)
            pltpu.sync_copy(ones_v, acc.at[cidx[q]], add=True)

            @pl.when(i + NB < N_CHUNKS)
            def _():
                cp_idx(i + NB, q)
        return carry

    lax.fori_loop(0, NFULL // NB, body, 0)
    for i in range(NFULL, N_CHUNKS):
        q = i % NB
        wait_idx(i, q)
        pltpu.sync_copy(ones_v, acc.at[cidx[q]], add=True)

    plsc.subcore_barrier()
    pltpu.sync_copy(
        acc.at[pl.ds(s * ROWS_PER_TILE, ROWS_PER_TILE)],
        out_hbm.at[c, pl.ds(s * ROWS_PER_TILE, ROWS_PER_TILE)],
    )


NI = 2 * NB


def _make_edge_scatter(d):
    @functools.partial(
        pl.kernel,
        out_type=jax.ShapeDtypeStruct((NC, N_PAD, d), jnp.float32),
        mesh=_MESH,
        scratch_types=(
            [pltpu.VMEM((2, CHUNK), jnp.int32) for _ in range(NI)]
            + [pltpu.VMEM((CHUNK, d), jnp.float32) for _ in range(NB)]
            + [pltpu.VMEM_SHARED((N_PAD, d), jnp.float32)]
            + [pltpu.SemaphoreType.DMA for _ in range(NI + 2 * NB)]
        ),
        compiler_params=_SC_PARAMS,
    )
    def edge_kernel(g_hbm, edge_hbm, zeros_hbm, out_hbm, *scr):
        idxb = scr[0:NI]
        rows = scr[NI:NI + NB]
        acc = scr[NI + NB]
        isem = scr[NI + NB + 1:2 * NI + NB + 1]
        gsem = scr[2 * NI + NB + 1:2 * NI + 2 * NB + 1]
        ssem = scr[2 * NI + 2 * NB + 1:2 * NI + 3 * NB + 1]
        c = lax.axis_index("c")
        s = lax.axis_index("s")
        wid = c * NS + s
        e0 = wid * E_PER_W
        pltpu.sync_copy(zeros_hbm,
                        acc.at[pl.ds(s * ROWS_PER_TILE, ROWS_PER_TILE)])
        plsc.subcore_barrier()

        def cp_idx(k, si):
            return pltpu.async_copy(
                edge_hbm.at[:, pl.ds(e0 + k * CHUNK, CHUNK)],
                idxb[si], isem[si])

        def wait_idx(k, si):
            pltpu.make_async_copy(
                edge_hbm.at[:, pl.ds(e0 + k * CHUNK, CHUNK)],
                idxb[si], isem[si]).wait()

        def start_gather(si, q):
            return pltpu.async_copy(g_hbm.at[idxb[si].at[0]], rows[q],
                                    gsem[q])

        def wait_gather(si, q):
            pltpu.make_async_copy(g_hbm.at[idxb[si].at[0]], rows[q],
                                  gsem[q]).wait()

        def start_scatter(si, q):
            return pltpu.async_copy(rows[q], acc.at[idxb[si].at[1]],
                                    ssem[q], add=True)

        def wait_scatter(si, q):
            pltpu.make_async_copy(rows[q], acc.at[idxb[si].at[1]],
                                  ssem[q]).wait()

        def maybe(pred, fn):
            def run():
                fn()
            if isinstance(pred, bool):
                if pred:
                    run()
            else:
                pl.when(pred)(run)

        for si in range(NI - GA):
            cp_idx(si, si)
        for k in range(GA):
            wait_idx(k, k)
            start_gather(k, k)

        def body_one(i, q, si):
            wait_gather(si, q)
            start_scatter(si, q)
            qn = (q + GA) % NB
            sn = (si - (NB - GA)) % NI
            maybe(i >= NB - GA, lambda: wait_scatter(sn, qn))
            maybe(i + NI - GA < N_CHUNKS,
                  lambda: cp_idx(i + NI - GA, (si - GA) % NI))

            def _gather_next():
                wait_idx(i + GA, (si + GA) % NI)
                start_gather((si + GA) % NI, qn)
            maybe(i + GA < N_CHUNKS, _gather_next)

        def outer(j0, carry):
            for k in range(NI):
                body_one(j0 * NI + k, k % NB, k % NI)
            return carry

        NOUTER = N_CHUNKS // NI
        lax.fori_loop(0, NOUTER, outer, 0)
        for i in range(NOUTER * NI, N_CHUNKS):
            body_one(i, i % NB, i % NI)
        for i in range(N_CHUNKS - (NB - GA), N_CHUNKS):
            wait_scatter(i % NI, i % NB)

        plsc.subcore_barrier()
        pltpu.sync_copy(
            acc.at[pl.ds(s * ROWS_PER_TILE, ROWS_PER_TILE)],
            out_hbm.at[c, pl.ds(s * ROWS_PER_TILE, ROWS_PER_TILE)],
        )

    return edge_kernel


_edge_scatter_128 = _make_edge_scatter(D_HID)
_edge_scatter_64 = _make_edge_scatter(D_OUT)


_BLK = 400


def _dis_body(degp_ref, dis_ref):
    deg = degp_ref[0, :, 0:1] + degp_ref[1, :, 0:1]
    dis_ref[...] = jnp.where(deg > 0, lax.rsqrt(deg), 0.0)


def _dis_kernel(deg_parts):
    return pl.pallas_call(
        _dis_body,
        grid=(8,),
        in_specs=[pl.BlockSpec((NC, N_PAD // 8, 16), lambda i: (0, i, 0))],
        out_specs=pl.BlockSpec((N_PAD // 8, 1), lambda i: (i, 0)),
        out_shape=jax.ShapeDtypeStruct((N_PAD, 1), jnp.float32),
    )(deg_parts)


def _pre_body(x_ref, wi_ref, wr_ref, b_ref, dis_ref, g_ref, r_ref):
    x = x_ref[...]
    h = jnp.dot(x, wi_ref[...], preferred_element_type=jnp.float32)
    g_ref[...] = h * dis_ref[...]
    r_ref[...] = (
        jnp.dot(x, wr_ref[...], preferred_element_type=jnp.float32)
        + b_ref[...]
    )


def _pre_kernel(x, wi, wr, b, dis):
    d_in, d = wi.shape
    return pl.pallas_call(
        _pre_body,
        grid=(N_NODES // _BLK,),
        in_specs=[
            pl.BlockSpec((_BLK, d_in), lambda i: (i, 0)),
            pl.BlockSpec((d_in, d), lambda i: (0, 0)),
            pl.BlockSpec((d_in, d), lambda i: (0, 0)),
            pl.BlockSpec((1, d), lambda i: (0, 0)),
            pl.BlockSpec((_BLK, 1), lambda i: (i, 0)),
        ],
        out_specs=[
            pl.BlockSpec((_BLK, d), lambda i: (i, 0)),
            pl.BlockSpec((_BLK, d), lambda i: (i, 0)),
        ],
        out_shape=[
            jax.ShapeDtypeStruct((N_NODES, d), jnp.float32),
            jax.ShapeDtypeStruct((N_NODES, d), jnp.float32),
        ],
    )(x, wi, wr, b, dis)


def _make_post_body(final):
    def body(parts_ref, r_ref, dis_ref, out_ref):
        z = (parts_ref[0] + parts_ref[1]) * dis_ref[...] + r_ref[...]
        z = jnp.maximum(z, 0.0)
        if final:
            z = jax.nn.sigmoid(z)
        out_ref[...] = z
    return body


def _post_kernel(parts, r, dis, final):
    d = r.shape[1]
    return pl.pallas_call(
        _make_post_body(final),
        grid=(N_NODES // _BLK,),
        in_specs=[
            pl.BlockSpec((NC, _BLK, d), lambda i: (0, i, 0)),
            pl.BlockSpec((_BLK, d), lambda i: (i, 0)),
            pl.BlockSpec((_BLK, 1), lambda i: (i, 0)),
        ],
        out_specs=pl.BlockSpec((_BLK, d), lambda i: (i, 0)),
        out_shape=jax.ShapeDtypeStruct((N_NODES, d), jnp.float32),
    )(parts, r, dis)


def kernel(x, edge_index, batch, W1_init, W1_root, b1,
           W2_init, W2_root, b2, W3_init, W3_root, b3):
    zeros128 = jnp.zeros((ROWS_PER_TILE, D_HID), jnp.float32)
    zeros64 = jnp.zeros((ROWS_PER_TILE, D_OUT), jnp.float32)
    zeros16 = jnp.zeros((ROWS_PER_TILE, 16), jnp.float32)
    ones16 = jnp.ones((CHUNK, 16), jnp.float32)

    deg_parts = _deg_kernel(edge_index, ones16, zeros16)
    dis = _dis_kernel(deg_parts)

    h = x
    for (wi, wr, b, scat, zeros, final) in (
        (W1_init, W1_root, b1, _edge_scatter_128, zeros128, False),
        (W2_init, W2_root, b2, _edge_scatter_128, zeros128, False),
        (W3_init, W3_root, b3, _edge_scatter_64, zeros64, True),
    ):
        g, r = _pre_kernel(h, wi, wr, jnp.reshape(b, (1, -1)), dis)
        parts = scat(g, edge_index, zeros)
        h = _post_kernel(parts, r, dis, final)
    return h


# speedup vs baseline: 20.7027x; 1.0751x over previous
import functools

import jax
import jax.numpy as jnp
from jax import lax
from jax.experimental import pallas as pl
from jax.experimental.pallas import tpu as pltpu
from jax.experimental.pallas import tpu_sc as plsc

N_NODES = 10000
N_PAD = 10240
E = 320000
D_IN = 128
D_HID = 128
D_OUT = 64

NC, NS = 2, 16
NW = NC * NS
E_PER_W = E // NW
CHUNK = 80
N_CHUNKS = E_PER_W // CHUNK
ROWS_PER_TILE = N_PAD // NS

_MESH = plsc.VectorSubcoreMesh(core_axis_name="c", subcore_axis_name="s")
_SC_PARAMS = pltpu.CompilerParams(use_tc_tiling_on_sc=False)


NB = 4
GA = 2
NFULL = (N_CHUNKS // NB) * NB


@functools.partial(
    pl.kernel,
    out_type=jax.ShapeDtypeStruct((NC, N_PAD, 16), jnp.float32),
    mesh=_MESH,
    scratch_types=(
        [pltpu.VMEM((CHUNK,), jnp.int32) for _ in range(NB)]
        + [pltpu.VMEM((CHUNK, 16), jnp.float32),
           pltpu.VMEM_SHARED((N_PAD, 16), jnp.float32)]
        + [pltpu.SemaphoreType.DMA for _ in range(NB)]
    ),
    compiler_params=_SC_PARAMS,
)
def _deg_kernel(edge_hbm, ones_hbm, zeros_hbm, out_hbm, *scr):
    cidx = scr[0:NB]
    ones_v, acc = scr[NB], scr[NB + 1]
    isem = scr[NB + 2:NB + 2 + NB]
    c = lax.axis_index("c")
    s = lax.axis_index("s")
    wid = c * NS + s
    e0 = wid * E_PER_W
    pltpu.sync_copy(zeros_hbm, acc.at[pl.ds(s * ROWS_PER_TILE, ROWS_PER_TILE)])
    pltpu.sync_copy(ones_hbm, ones_v)
    plsc.subcore_barrier()

    def cp_idx(k, q):
        return pltpu.async_copy(
            edge_hbm.at[1, pl.ds(e0 + k * CHUNK, CHUNK)], cidx[q], isem[q])

    def wait_idx(k, q):
        pltpu.make_async_copy(
            edge_hbm.at[1, pl.ds(e0 + k * CHUNK, CHUNK)],
            cidx[q], isem[q]).wait()

    for q in range(NB):
        cp_idx(q, q)

    def body(i0, carry):
        for q in range(NB):
            i = i0 * NB + q
            wait_idx(i, q)
            pltpu.sync_copy(ones_v, acc.at[cidx[q]], add=True)

            @pl.when(i + NB < N_CHUNKS)
            def _():
                cp_idx(i + NB, q)
        return carry

    lax.fori_loop(0, NFULL // NB, body, 0)
    for i in range(NFULL, N_CHUNKS):
        q = i % NB
        wait_idx(i, q)
        pltpu.sync_copy(ones_v, acc.at[cidx[q]], add=True)

    plsc.subcore_barrier()
    pltpu.sync_copy(
        acc.at[pl.ds(s * ROWS_PER_TILE, ROWS_PER_TILE)],
        out_hbm.at[c, pl.ds(s * ROWS_PER_TILE, ROWS_PER_TILE)],
    )


NI = 2 * NB


def _make_edge_scatter(d):
    @functools.partial(
        pl.kernel,
        out_type=jax.ShapeDtypeStruct((NC, N_PAD, d), jnp.float32),
        mesh=_MESH,
        scratch_types=(
            [pltpu.VMEM((2, CHUNK), jnp.int32) for _ in range(NI)]
            + [pltpu.VMEM((CHUNK, d), jnp.float32) for _ in range(NB)]
            + [pltpu.VMEM_SHARED((N_PAD, d), jnp.float32)]
            + [pltpu.SemaphoreType.DMA for _ in range(NI + 2 * NB)]
        ),
        compiler_params=_SC_PARAMS,
    )
    def edge_kernel(g_hbm, edge_hbm, zeros_hbm, out_hbm, *scr):
        idxb = scr[0:NI]
        rows = scr[NI:NI + NB]
        acc = scr[NI + NB]
        isem = scr[NI + NB + 1:2 * NI + NB + 1]
        gsem = scr[2 * NI + NB + 1:2 * NI + 2 * NB + 1]
        ssem = scr[2 * NI + 2 * NB + 1:2 * NI + 3 * NB + 1]
        c = lax.axis_index("c")
        s = lax.axis_index("s")
        wid = c * NS + s
        e0 = wid * E_PER_W
        pltpu.sync_copy(zeros_hbm,
                        acc.at[pl.ds(s * ROWS_PER_TILE, ROWS_PER_TILE)])
        plsc.subcore_barrier()

        def cp_idx(k, si):
            return pltpu.async_copy(
                edge_hbm.at[:, pl.ds(e0 + k * CHUNK, CHUNK)],
                idxb[si], isem[si])

        def wait_idx(k, si):
            pltpu.make_async_copy(
                edge_hbm.at[:, pl.ds(e0 + k * CHUNK, CHUNK)],
                idxb[si], isem[si]).wait()

        def start_gather(si, q):
            return pltpu.async_copy(g_hbm.at[idxb[si].at[0]], rows[q],
                                    gsem[q])

        def wait_gather(si, q):
            pltpu.make_async_copy(g_hbm.at[idxb[si].at[0]], rows[q],
                                  gsem[q]).wait()

        def start_scatter(si, q):
            return pltpu.async_copy(rows[q], acc.at[idxb[si].at[1]],
                                    ssem[q], add=True)

        def wait_scatter(si, q):
            pltpu.make_async_copy(rows[q], acc.at[idxb[si].at[1]],
                                  ssem[q]).wait()

        def maybe(pred, fn):
            def run():
                fn()
            if isinstance(pred, bool):
                if pred:
                    run()
            else:
                pl.when(pred)(run)

        for si in range(NI - GA):
            cp_idx(si, si)
        for k in range(GA):
            wait_idx(k, k)
            start_gather(k, k)

        def body_one(i, q, si):
            wait_gather(si, q)
            start_scatter(si, q)
            qn = (q + GA) % NB
            sn = (si - (NB - GA)) % NI
            maybe(i >= NB - GA, lambda: wait_scatter(sn, qn))
            maybe(i + NI - GA < N_CHUNKS,
                  lambda: cp_idx(i + NI - GA, (si - GA) % NI))

            def _gather_next():
                wait_idx(i + GA, (si + GA) % NI)
                start_gather((si + GA) % NI, qn)
            maybe(i + GA < N_CHUNKS, _gather_next)

        def outer(j0, carry):
            for k in range(NI):
                body_one(j0 * NI + k, k % NB, k % NI)
            return carry

        NOUTER = N_CHUNKS // NI
        lax.fori_loop(0, NOUTER, outer, 0)
        for i in range(NOUTER * NI, N_CHUNKS):
            body_one(i, i % NB, i % NI)
        for i in range(N_CHUNKS - (NB - GA), N_CHUNKS):
            wait_scatter(i % NI, i % NB)

        plsc.subcore_barrier()
        pltpu.sync_copy(
            acc.at[pl.ds(s * ROWS_PER_TILE, ROWS_PER_TILE)],
            out_hbm.at[c, pl.ds(s * ROWS_PER_TILE, ROWS_PER_TILE)],
        )

    return edge_kernel


_edge_scatter_128 = _make_edge_scatter(D_HID)
_edge_scatter_64 = _make_edge_scatter(D_OUT)


_BLK = 400


def _dis_block(degp_ref):
    deg = degp_ref[0, :, 0:1] + degp_ref[1, :, 0:1]
    return jnp.where(deg > 0, lax.rsqrt(deg), 0.0)


def _pre_body(x_ref, degp_ref, wi_ref, wr_ref, b_ref, g_ref, r_ref):
    dis = _dis_block(degp_ref)
    x = x_ref[...]
    h = jnp.dot(x, wi_ref[...], preferred_element_type=jnp.float32)
    g_ref[...] = h * dis
    r_ref[...] = (
        jnp.dot(x, wr_ref[...], preferred_element_type=jnp.float32)
        + b_ref[...]
    )


def _pre_kernel(x, deg_parts, wi, wr, b):
    d_in, d = wi.shape
    return pl.pallas_call(
        _pre_body,
        grid=(N_NODES // _BLK,),
        in_specs=[
            pl.BlockSpec((_BLK, d_in), lambda i: (i, 0)),
            pl.BlockSpec((NC, _BLK, 16), lambda i: (0, i, 0)),
            pl.BlockSpec((d_in, d), lambda i: (0, 0)),
            pl.BlockSpec((d_in, d), lambda i: (0, 0)),
            pl.BlockSpec((1, d), lambda i: (0, 0)),
        ],
        out_specs=[
            pl.BlockSpec((_BLK, d), lambda i: (i, 0)),
            pl.BlockSpec((_BLK, d), lambda i: (i, 0)),
        ],
        out_shape=[
            jax.ShapeDtypeStruct((N_NODES, d), jnp.float32),
            jax.ShapeDtypeStruct((N_NODES, d), jnp.float32),
        ],
    )(x, deg_parts, wi, wr, b)


def _mid_body(parts_ref, r_ref, degp_ref, wi_ref, wr_ref, b_ref,
              g_ref, rn_ref):
    dis = _dis_block(degp_ref)
    out = jnp.maximum(
        (parts_ref[0] + parts_ref[1]) * dis + r_ref[...], 0.0)
    h = jnp.dot(out, wi_ref[...], preferred_element_type=jnp.float32)
    g_ref[...] = h * dis
    rn_ref[...] = (
        jnp.dot(out, wr_ref[...], preferred_element_type=jnp.float32)
        + b_ref[...]
    )


def _mid_kernel(parts, r, deg_parts, wi, wr, b):
    d_in, d = wi.shape
    return pl.pallas_call(
        _mid_body,
        grid=(N_NODES // _BLK,),
        in_specs=[
            pl.BlockSpec((NC, _BLK, d_in), lambda i: (0, i, 0)),
            pl.BlockSpec((_BLK, d_in), lambda i: (i, 0)),
            pl.BlockSpec((NC, _BLK, 16), lambda i: (0, i, 0)),
            pl.BlockSpec((d_in, d), lambda i: (0, 0)),
            pl.BlockSpec((d_in, d), lambda i: (0, 0)),
            pl.BlockSpec((1, d), lambda i: (0, 0)),
        ],
        out_specs=[
            pl.BlockSpec((_BLK, d), lambda i: (i, 0)),
            pl.BlockSpec((_BLK, d), lambda i: (i, 0)),
        ],
        out_shape=[
            jax.ShapeDtypeStruct((N_NODES, d), jnp.float32),
            jax.ShapeDtypeStruct((N_NODES, d), jnp.float32),
        ],
    )(parts, r, deg_parts, wi, wr, b)


def _post_body(parts_ref, r_ref, degp_ref, out_ref):
    dis = _dis_block(degp_ref)
    z = (parts_ref[0] + parts_ref[1]) * dis + r_ref[...]
    out_ref[...] = jax.nn.sigmoid(jnp.maximum(z, 0.0))


def _post_kernel(parts, r, deg_parts):
    d = r.shape[1]
    return pl.pallas_call(
        _post_body,
        grid=(N_NODES // _BLK,),
        in_specs=[
            pl.BlockSpec((NC, _BLK, d), lambda i: (0, i, 0)),
            pl.BlockSpec((_BLK, d), lambda i: (i, 0)),
            pl.BlockSpec((NC, _BLK, 16), lambda i: (0, i, 0)),
        ],
        out_specs=pl.BlockSpec((_BLK, d), lambda i: (i, 0)),
        out_shape=jax.ShapeDtypeStruct((N_NODES, d), jnp.float32),
    )(parts, r, deg_parts)


def kernel(x, edge_index, batch, W1_init, W1_root, b1,
           W2_init, W2_root, b2, W3_init, W3_root, b3):
    zeros128 = jnp.zeros((ROWS_PER_TILE, D_HID), jnp.float32)
    zeros64 = jnp.zeros((ROWS_PER_TILE, D_OUT), jnp.float32)
    zeros16 = jnp.zeros((ROWS_PER_TILE, 16), jnp.float32)
    ones16 = jnp.ones((CHUNK, 16), jnp.float32)

    deg_parts = _deg_kernel(edge_index, ones16, zeros16)

    g, r = _pre_kernel(x, deg_parts, W1_init, W1_root,
                       jnp.reshape(b1, (1, -1)))
    parts = _edge_scatter_128(g, edge_index, zeros128)
    g, r = _mid_kernel(parts, r, deg_parts, W2_init, W2_root,
                       jnp.reshape(b2, (1, -1)))
    parts = _edge_scatter_128(g, edge_index, zeros128)
    g, r = _mid_kernel(parts, r, deg_parts, W3_init, W3_root,
                       jnp.reshape(b3, (1, -1)))
    parts = _edge_scatter_64(g, edge_index, zeros64)
    return _post_kernel(parts, r, deg_parts)
